# Initial kernel scaffold; baseline (speedup 1.0000x reference)
import jax
import jax.numpy as jnp
from jax.experimental import pallas as pl


def kernel(sdf, all_edges):
    raise NotImplementedError("write your pallas kernel here")


# speedup vs baseline: 25.5530x; 25.5530x over previous
import functools

import jax
import jax.numpy as jnp
from jax import lax
from jax.experimental import pallas as pl
from jax.experimental.pallas import tpu as pltpu
from jax.experimental.pallas import tpu_sc as plsc

_N_VERTS = 2146689
_N_EDGES = 6390784
_NC = 2
_NS = 16
_NW = _NC * _NS
_EPW = _N_EDGES // _NW
_CHUNK = 2528
_NCHUNK = _EPW // _CHUNK
_CW = 2 * _CHUNK
_GROUPS = _CHUNK // 16


def _sc_partials(edges_flat, sdf):
    mesh = plsc.VectorSubcoreMesh(core_axis_name="c", subcore_axis_name="s")

    @functools.partial(
        pl.kernel,
        mesh=mesh,
        compiler_params=pltpu.CompilerParams(needs_layout_passes=False),
        out_type=jax.ShapeDtypeStruct((_NW, 16), jnp.float32),
        scratch_types=[
            pltpu.VMEM((_CW,), jnp.int32),
            pltpu.VMEM((_CW,), jnp.float32),
            pltpu.VMEM((16,), jnp.float32),
            pltpu.SemaphoreType.DMA,
        ],
    )
    def body(edges_hbm, sdf_hbm, out_hbm, idx_v, val_v, res_v, sem):
        wid = lax.axis_index("s") * _NC + lax.axis_index("c")
        lane = lax.broadcasted_iota(jnp.int32, (16,), 0)
        zeros = jnp.zeros((16,), jnp.float32)
        ones = jnp.ones((16,), jnp.float32)

        def chunk_body(k, carry):
            s_acc, c_acc = carry
            woff = wid * (2 * _EPW) + k * _CW
            pltpu.sync_copy(edges_hbm.at[pl.ds(woff, _CW)], idx_v)
            pltpu.async_copy(sdf_hbm.at[idx_v], val_v, sem).wait()

            def group_body(j, gc):
                s, c = gc
                ia = j * 32 + 2 * lane
                a = plsc.load_gather(val_v, [ia])
                b = plsc.load_gather(val_v, [ia + 1])
                mask = jnp.sign(a) != jnp.sign(b)
                aa = jnp.abs(a)
                ab = jnp.abs(b)
                ua = jnp.exp(-aa)
                ub = jnp.exp(-ab)
                za = ua / (ua + 2.0)
                zb = ub / (ub + 2.0)
                za2 = za * za
                zb2 = zb * zb
                spa = za * (2.0 + za2 * (2.0 / 3.0 + za2 * (2.0 / 5.0 + za2 * (2.0 / 7.0))))
                spb = zb * (2.0 + zb2 * (2.0 / 3.0 + zb2 * (2.0 / 5.0 + zb2 * (2.0 / 7.0))))
                y1 = (b > 0.0).astype(jnp.float32)
                y2 = (a > 0.0).astype(jnp.float32)
                term = (jnp.maximum(a, 0.0) - a * y1 + spa
                        + jnp.maximum(b, 0.0) - b * y2 + spb)
                s = s + jnp.where(mask, term, zeros)
                c = c + jnp.where(mask, ones, zeros)
                return s, c

            return lax.fori_loop(0, _GROUPS, group_body, (s_acc, c_acc))

        s_acc, c_acc = lax.fori_loop(
            0, _NCHUNK, chunk_body, (zeros, zeros))
        s_tot = jnp.sum(s_acc)
        c_tot = jnp.sum(c_acc)
        res = jnp.where(lane == 0, s_tot, jnp.where(lane == 1, c_tot, 0.0))
        res_v[...] = res
        pltpu.sync_copy(res_v, out_hbm.at[wid])

    return body(edges_flat, sdf)


def _finish(partials):
    def body(p_ref, o_ref):
        x = p_ref[...]
        s = jnp.sum(x[:, 0])
        c = jnp.sum(x[:, 1])
        o_ref[0] = s / jnp.maximum(c, 1.0)

    return pl.pallas_call(
        body,
        out_shape=jax.ShapeDtypeStruct((1,), jnp.float32),
        out_specs=pl.BlockSpec(memory_space=pltpu.SMEM),
    )(partials)


def kernel(sdf, all_edges):
    edges_flat = all_edges.reshape(-1)
    partials = _sc_partials(edges_flat, sdf)
    return _finish(partials)[0]


# speedup vs baseline: 26.2812x; 1.0285x over previous
import functools

import jax
import jax.numpy as jnp
from jax import lax
from jax.experimental import pallas as pl
from jax.experimental.pallas import tpu as pltpu
from jax.experimental.pallas import tpu_sc as plsc

_N_VERTS = 2146689
_SH2 = 67328
_NV2P = 16 * _SH2
_NV_PAD = 2 * _NV2P
_N_EDGES = 6390784
_NC = 2
_NS = 16
_NW = _NC * _NS
_CHUNK = 1024
_NCH_TOT = _N_EDGES // _CHUNK
_CW = 2 * _CHUNK
_GROUPS = _CHUNK // 16
_CGRP = _CW // 16


def _sc_partials(edges_flat, table32):
    mesh = plsc.VectorSubcoreMesh(core_axis_name="c", subcore_axis_name="s")

    @functools.partial(
        pl.kernel,
        mesh=mesh,
        compiler_params=pltpu.CompilerParams(needs_layout_passes=False),
        out_type=jax.ShapeDtypeStruct((_NW, 16), jnp.float32),
        scratch_types=[
            pltpu.VMEM((3 * _CW,), jnp.int32),
            pltpu.VMEM((2 * _CW,), jnp.int32),
            pltpu.VMEM((2 * _CW,), jnp.int32),
            pltpu.VMEM((16,), jnp.float32),
            pltpu.VMEM_SHARED((_NV2P,), jnp.int32),
            pltpu.SemaphoreType.DMA((3,)),
            pltpu.SemaphoreType.DMA((2,)),
        ],
    )
    def body(edges_hbm, tab_hbm, out_hbm, idx_v, hidx_v, val_v,
             res_v, table, sem_i, sem_g):
        cid = lax.axis_index("c")
        sid = lax.axis_index("s")
        wid = sid * _NC + cid
        n_k = 195 + (wid == 0).astype(jnp.int32)
        lane = lax.broadcasted_iota(jnp.int32, (16,), 0)
        zeros = jnp.zeros((16,), jnp.float32)
        ones = jnp.ones((16,), jnp.float32)

        pltpu.sync_copy(tab_hbm.at[pl.ds(sid * _SH2, _SH2)],
                        table.at[pl.ds(sid * _SH2, _SH2)])
        plsc.subcore_barrier()

        def _idx_args(k):
            b = lax.rem(k, 3)
            return (edges_hbm.at[pl.ds((wid + _NW * k) * _CW, _CW)],
                    idx_v.at[pl.ds(b * _CW, _CW)], sem_i.at[b])

        def _gat_args(k):
            b = lax.rem(k, 2)
            return (table.at[hidx_v.at[pl.ds(b * _CW, _CW)]],
                    val_v.at[pl.ds(b * _CW, _CW)], sem_g.at[b])

        def idx_start(k):
            return pltpu.async_copy(*_idx_args(k))

        def idx_wait(k):
            pltpu.make_async_copy(*_idx_args(k)).wait()

        def gather_start(k):
            return pltpu.async_copy(*_gat_args(k))

        def gather_wait(k):
            pltpu.make_async_copy(*_gat_args(k)).wait()

        def halve_pass(k):
            b3 = lax.rem(k, 3) * _CW
            b2 = lax.rem(k, 2) * _CW

            def cbody(g, _):
                v = idx_v[pl.ds(b3 + g * 16, 16)]
                hidx_v[pl.ds(b2 + g * 16, 16)] = lax.shift_right_logical(v, 1)
                return 0

            lax.fori_loop(0, _CGRP, cbody, 0)

        idx_start(0).wait()
        halve_pass(0)
        gather_start(0)
        idx_start(1)

        def chunk_body(k, carry):
            s_acc, c_acc = carry

            @pl.when(k + 1 < n_k)
            def _():
                idx_wait(k + 1)
                halve_pass(k + 1)
                gather_start(k + 1)

            gather_wait(k)

            @pl.when(k + 2 < n_k)
            def _():
                idx_start(k + 2)

            vref = val_v.at[pl.ds(lax.rem(k, 2) * _CW, _CW)]
            iref = idx_v.at[pl.ds(lax.rem(k, 3) * _CW, _CW)]

            def group_body(j, gc):
                s, c = gc
                ie = j * 32 + 2 * lane
                io = ie + 1

                def fetch(ix):
                    w = plsc.load_gather(vref, [ix])
                    orig = plsc.load_gather(iref, [ix])
                    odd = lax.bitwise_and(orig, 1) == 1
                    bits = jnp.where(odd, lax.bitwise_and(w, -65536),
                                     lax.shift_left(w, 16))
                    return plsc.bitcast(bits, jnp.float32)

                a = fetch(ie)
                b = fetch(io)
                mask = jnp.sign(a) != jnp.sign(b)
                aa = jnp.abs(a)
                ab = jnp.abs(b)
                ua = jnp.exp(-aa)
                ub = jnp.exp(-ab)
                za = ua / (ua + 2.0)
                zb = ub / (ub + 2.0)
                za2 = za * za
                zb2 = zb * zb
                spa = za * (2.0 + za2 * (2.0 / 3.0 + za2 * (2.0 / 5.0 + za2 * (2.0 / 7.0))))
                spb = zb * (2.0 + zb2 * (2.0 / 3.0 + zb2 * (2.0 / 5.0 + zb2 * (2.0 / 7.0))))
                y1 = (b > 0.0).astype(jnp.float32)
                y2 = (a > 0.0).astype(jnp.float32)
                term = (jnp.maximum(a, 0.0) - a * y1 + spa
                        + jnp.maximum(b, 0.0) - b * y2 + spb)
                s = s + jnp.where(mask, term, zeros)
                c = c + jnp.where(mask, ones, zeros)
                return s, c

            return lax.fori_loop(0, _GROUPS, group_body, (s_acc, c_acc))

        s_acc, c_acc = lax.fori_loop(0, n_k, chunk_body, (zeros, zeros))
        s_tot = jnp.sum(s_acc)
        c_tot = jnp.sum(c_acc)
        res = jnp.where(lane == 0, s_tot, jnp.where(lane == 1, c_tot, 0.0))
        res_v[...] = res
        pltpu.sync_copy(res_v, out_hbm.at[wid])

    return body(edges_flat, table32)


def _finish(partials):
    def body(p_ref, o_ref):
        x = p_ref[...]
        s = jnp.sum(x[:, 0])
        c = jnp.sum(x[:, 1])
        o_ref[0] = s / jnp.maximum(c, 1.0)

    return pl.pallas_call(
        body,
        out_shape=jax.ShapeDtypeStruct((1,), jnp.float32),
        out_specs=pl.BlockSpec(memory_space=pltpu.SMEM),
    )(partials)


def kernel(sdf, all_edges):
    edges_flat = all_edges.reshape(-1)
    sdf16 = jnp.pad(sdf, (0, _NV_PAD - _N_VERTS)).astype(jnp.bfloat16)
    table32 = lax.bitcast_convert_type(sdf16.reshape(-1, 2), jnp.int32)
    partials = _sc_partials(edges_flat, table32)
    return _finish(partials)[0]


# speedup vs baseline: 645.8026x; 24.5728x over previous
import functools

import jax
import jax.numpy as jnp
from jax import lax
from jax.experimental import pallas as pl
from jax.experimental.pallas import tpu as pltpu
from jax.experimental.pallas import tpu_sc as plsc

_N_VERTS = 2146689
_SH2 = 67584
_NV2P = 16 * _SH2
_NV_PAD = 2 * _NV2P
_STG = _NV_PAD // 16
_SUB = 8
_TMPW = _STG // _SUB
_PKW = _TMPW // 2
_N_EDGES = 6390784
_NC = 2
_NS = 16
_NW = _NC * _NS
_CHUNK = 1024
_CW = 2 * _CHUNK
_GROUPS = _CHUNK // 16
_NKBASE = 195


def _sc_partials(tcat, rb):
    mesh = plsc.VectorSubcoreMesh(core_axis_name="c", subcore_axis_name="s")

    @functools.partial(
        pl.kernel,
        mesh=mesh,
        compiler_params=pltpu.CompilerParams(needs_layout_passes=False),
        out_type=jax.ShapeDtypeStruct((_NW, 16), jnp.float32),
        scratch_types=[
            pltpu.VMEM((_TMPW,), jnp.int32),
            pltpu.VMEM((_PKW,), jnp.int32),
            pltpu.VMEM((3 * _CW,), jnp.int32),
            pltpu.VMEM((2 * _CW,), jnp.int32),
            pltpu.VMEM((2 * _CW,), jnp.int32),
            pltpu.VMEM((16,), jnp.float32),
            pltpu.VMEM_SHARED((_NV2P,), jnp.int32),
            pltpu.SemaphoreType.DMA((3,)),
            pltpu.SemaphoreType.DMA((2,)),
        ],
    )
    def body(tcat_hbm, rb_hbm, out_hbm, tmp_v, pack_v, idx_v,
             hidx_v, val_v, res_v, table, sem_i, sem_g):
        cid = lax.axis_index("c")
        sid = lax.axis_index("s")
        wid = sid * _NC + cid
        n_k = _NKBASE + (wid == 0).astype(jnp.int32)
        lane = lax.broadcasted_iota(jnp.int32, (16,), 0)
        lane2 = 2 * lane
        zeros = jnp.zeros((16,), jnp.float32)
        ones = jnp.ones((16,), jnp.float32)

        for q in range(_SUB):
            pltpu.sync_copy(
                rb_hbm.at[pl.ds(sid * _STG + q * _TMPW, _TMPW)], tmp_v)

            def pk(g, _):
                base = g * 32
                we = plsc.load_gather(tmp_v, [base + lane2])
                wo = plsc.load_gather(tmp_v, [base + lane2 + 1])
                pack_v[pl.ds(g * 16, 16)] = lax.bitwise_or(
                    we, lax.shift_left(wo, 16))
                return 0

            lax.fori_loop(0, _PKW // 16, pk, 0)
            pltpu.sync_copy(
                pack_v, table.at[pl.ds(sid * _SH2 + q * _PKW, _PKW)])
        plsc.subcore_barrier()

        def _idx_args(k):
            b = lax.rem(k, 3)
            return (tcat_hbm.at[pl.ds((wid + _NW * k) * _CW, _CW)],
                    idx_v.at[pl.ds(b * _CW, _CW)], sem_i.at[b])

        def _gat_args(k):
            b = lax.rem(k, 2)
            return (table.at[hidx_v.at[pl.ds(b * _CW, _CW)]],
                    val_v.at[pl.ds(b * _CW, _CW)], sem_g.at[b])

        def idx_start(k):
            pltpu.async_copy(*_idx_args(k))

        def idx_wait(k):
            pltpu.make_async_copy(*_idx_args(k)).wait()

        def gather_start(k):
            pltpu.async_copy(*_gat_args(k))

        def gather_wait(k):
            pltpu.make_async_copy(*_gat_args(k)).wait()

        def mask_pass(k):
            b3 = lax.rem(k, 3) * _CW
            b2 = lax.rem(k, 2) * _CW

            def cbody(g, _):
                o = g * 16
                v = idx_v[pl.ds(b3 + o, 16)]
                hidx_v[pl.ds(b2 + o, 16)] = lax.bitwise_and(v, 0x7FFFFFFF)
                return 0

            lax.fori_loop(0, 2 * _GROUPS, cbody, 0)

        idx_start(0)
        idx_wait(0)
        mask_pass(0)
        gather_start(0)
        idx_start(1)

        def chunk_body(k, carry):
            s_acc, c_acc = carry

            @pl.when(k + 1 < n_k)
            def _():
                idx_wait(k + 1)
                mask_pass(k + 1)
                gather_start(k + 1)

            gather_wait(k)

            @pl.when(k + 2 < n_k)
            def _():
                idx_start(k + 2)

            b3 = lax.rem(k, 3) * _CW
            b2 = lax.rem(k, 2) * _CW

            def group_body(j, gc):
                s, c = gc
                ao = (j // 8) * 256 + (j % 8) * 16
                bo = ao + 128

                def fetch(o):
                    w = val_v[pl.ds(b2 + o, 16)]
                    t = idx_v[pl.ds(b3 + o, 16)]
                    odd = t < 0
                    bits = jnp.where(odd, lax.bitwise_and(w, -65536),
                                     lax.shift_left(w, 16))
                    return plsc.bitcast(bits, jnp.float32)

                a = fetch(ao)
                b = fetch(bo)
                ga = a > 0.0
                gb = b > 0.0
                mask = ga != gb
                ua = jnp.exp(-jnp.abs(a))
                ub = jnp.exp(-jnp.abs(b))
                spa = ua * (0.9992355 + ua * (-0.49023072 + ua * (0.28527268 + ua * (-0.13158183 + ua * 0.030449))))
                spb = ub * (0.9992355 + ub * (-0.49023072 + ub * (0.28527268 + ub * (-0.13158183 + ub * 0.030449))))
                y1 = gb.astype(jnp.float32)
                y2 = ga.astype(jnp.float32)
                term = (jnp.maximum(a, 0.0) - a * y1 + spa
                        + jnp.maximum(b, 0.0) - b * y2 + spb)
                s = s + jnp.where(mask, term, zeros)
                c = c + jnp.where(mask, ones, zeros)
                return s, c

            return lax.fori_loop(0, _GROUPS, group_body, (s_acc, c_acc))

        s_acc, c_acc = lax.fori_loop(0, n_k, chunk_body, (zeros, zeros))
        s_tot = jnp.sum(s_acc)
        c_tot = jnp.sum(c_acc)
        res = jnp.where(lane == 0, s_tot, jnp.where(lane == 1, c_tot, 0.0))
        res_v[...] = res
        pltpu.sync_copy(res_v, out_hbm.at[wid])

    return body(tcat, rb)


def _finish(partials):
    def body(p_ref, o_ref):
        x = p_ref[...]
        s = jnp.sum(x[:, 0])
        c = jnp.sum(x[:, 1])
        o_ref[0] = s / jnp.maximum(c, 1.0)

    return pl.pallas_call(
        body,
        out_shape=jax.ShapeDtypeStruct((1,), jnp.float32),
        out_specs=pl.BlockSpec(memory_space=pltpu.SMEM),
    )(partials)


def kernel(sdf, all_edges):
    tcat = all_edges.reshape(-1, 128, 2).transpose(0, 2, 1).reshape(-1)
    tpk = lax.bitwise_or(lax.shift_right_logical(tcat, 1),
                         lax.shift_left(lax.bitwise_and(tcat, 1), 31))
    bits = lax.bitcast_convert_type(
        jnp.pad(sdf, (0, _NV_PAD - _N_VERTS)), jnp.uint32)
    rb = jnp.right_shift(
        bits + jnp.uint32(0x7FFF)
        + jnp.bitwise_and(jnp.right_shift(bits, jnp.uint32(16)),
                          jnp.uint32(1)),
        jnp.uint32(16)).astype(jnp.int32)
    partials = _sc_partials(tpk, rb)
    return _finish(partials)[0]


# speedup vs baseline: 683.0794x; 1.0577x over previous
import functools

import jax
import jax.numpy as jnp
from jax import lax
from jax.experimental import pallas as pl
from jax.experimental.pallas import tpu as pltpu
from jax.experimental.pallas import tpu_sc as plsc

_N_VERTS = 2146689
_SH2 = 67584
_NV2P = 16 * _SH2
_NV_PAD = 2 * _NV2P
_STG = _NV_PAD // 16
_SUB = 8
_TMPW = _STG // _SUB
_PKW = _TMPW // 2
_N_EDGES = 6390784
_NC = 2
_NS = 16
_NW = _NC * _NS
_CHUNK = 2048
_CW = 2 * _CHUNK
_GROUPS = _CHUNK // 16
_NCH = 3121
_LASTOFF = 2 * _N_EDGES - _CW
_NKBASE = 97


def _sc_partials(tcat, rb):
    mesh = plsc.VectorSubcoreMesh(core_axis_name="c", subcore_axis_name="s")

    @functools.partial(
        pl.kernel,
        mesh=mesh,
        compiler_params=pltpu.CompilerParams(needs_layout_passes=False),
        out_type=jax.ShapeDtypeStruct((_NW, 16), jnp.float32),
        scratch_types=[
            pltpu.VMEM((_TMPW,), jnp.int32),
            pltpu.VMEM((_PKW,), jnp.int32),
            pltpu.VMEM((3 * _CW,), jnp.int32),
            pltpu.VMEM((2 * _CW,), jnp.int32),
            pltpu.VMEM((2 * _CW,), jnp.int32),
            pltpu.VMEM((16,), jnp.float32),
            pltpu.VMEM_SHARED((_NV2P,), jnp.int32),
            pltpu.SemaphoreType.DMA((3,)),
            pltpu.SemaphoreType.DMA((2,)),
        ],
    )
    def body(tcat_hbm, rb_hbm, out_hbm, tmp_v, pack_v, idx_v,
             hidx_v, val_v, res_v, table, sem_i, sem_g):
        cid = lax.axis_index("c")
        sid = lax.axis_index("s")
        wid = sid * _NC + cid
        n_k = _NKBASE + (wid < 17).astype(jnp.int32)
        lane = lax.broadcasted_iota(jnp.int32, (16,), 0)
        lane2 = 2 * lane
        zeros = jnp.zeros((16,), jnp.float32)
        ones = jnp.ones((16,), jnp.float32)

        for q in range(_SUB):
            pltpu.sync_copy(
                rb_hbm.at[pl.ds(sid * _STG + q * _TMPW, _TMPW)], tmp_v)

            def pk(g, _):
                base = g * 32
                we = plsc.load_gather(tmp_v, [base + lane2])
                wo = plsc.load_gather(tmp_v, [base + lane2 + 1])
                pack_v[pl.ds(g * 16, 16)] = lax.bitwise_or(
                    we, lax.shift_left(wo, 16))
                return 0

            lax.fori_loop(0, _PKW // 16, pk, 0, unroll=4)
            pltpu.sync_copy(
                pack_v, table.at[pl.ds(sid * _SH2 + q * _PKW, _PKW)])
        plsc.subcore_barrier()

        def _idx_args(k):
            b = lax.rem(k, 3)
            off = jnp.minimum((wid + _NW * k) * _CW, _LASTOFF)
            return (tcat_hbm.at[pl.ds(off, _CW)],
                    idx_v.at[pl.ds(b * _CW, _CW)], sem_i.at[b])

        def _gat_args(k):
            b = lax.rem(k, 2)
            return (table.at[hidx_v.at[pl.ds(b * _CW, _CW)]],
                    val_v.at[pl.ds(b * _CW, _CW)], sem_g.at[b])

        def idx_start(k):
            pltpu.async_copy(*_idx_args(k))

        def idx_wait(k):
            pltpu.make_async_copy(*_idx_args(k)).wait()

        def gather_start(k):
            pltpu.async_copy(*_gat_args(k))

        def gather_wait(k):
            pltpu.make_async_copy(*_gat_args(k)).wait()

        def mask_pass(k):
            b3 = lax.rem(k, 3) * _CW
            b2 = lax.rem(k, 2) * _CW

            def cbody(g, _):
                o = g * 16
                v = idx_v[pl.ds(b3 + o, 16)]
                hidx_v[pl.ds(b2 + o, 16)] = lax.bitwise_and(v, 0x7FFFFFFF)
                return 0

            lax.fori_loop(0, 2 * _GROUPS, cbody, 0, unroll=8)

        idx_start(0)
        idx_wait(0)
        mask_pass(0)
        gather_start(0)
        idx_start(1)

        def chunk_body(k, carry):
            s_acc, c_acc = carry

            @pl.when(k + 1 < n_k)
            def _():
                idx_wait(k + 1)
                mask_pass(k + 1)
                gather_start(k + 1)

            gather_wait(k)

            @pl.when(k + 2 < n_k)
            def _():
                idx_start(k + 2)

            b3 = lax.rem(k, 3) * _CW
            b2 = lax.rem(k, 2) * _CW
            lim = jnp.where(wid + _NW * k == _NCH - 1, _CHUNK // 2, 0)

            def group_body(j, gc):
                s, c = gc
                ao = (j // 8) * 256 + (j % 8) * 16
                bo = ao + 128
                eid = (j // 8) * 128 + (j % 8) * 16 + lane

                def fetch(o):
                    w = val_v[pl.ds(b2 + o, 16)]
                    t = idx_v[pl.ds(b3 + o, 16)]
                    odd = t < 0
                    bits = jnp.where(odd, lax.bitwise_and(w, -65536),
                                     lax.shift_left(w, 16))
                    return plsc.bitcast(bits, jnp.float32)

                a = fetch(ao)
                b = fetch(bo)
                ga = a > 0.0
                gb = b > 0.0
                mask = jnp.logical_and(ga != gb, eid >= lim)
                ua = jnp.exp(-jnp.abs(a))
                ub = jnp.exp(-jnp.abs(b))
                spa = ua * (0.9992355 + ua * (-0.49023072 + ua * (0.28527268 + ua * (-0.13158183 + ua * 0.030449))))
                spb = ub * (0.9992355 + ub * (-0.49023072 + ub * (0.28527268 + ub * (-0.13158183 + ub * 0.030449))))
                y1 = gb.astype(jnp.float32)
                y2 = ga.astype(jnp.float32)
                term = (jnp.maximum(a, 0.0) - a * y1 + spa
                        + jnp.maximum(b, 0.0) - b * y2 + spb)
                s = s + jnp.where(mask, term, zeros)
                c = c + jnp.where(mask, ones, zeros)
                return s, c

            return lax.fori_loop(0, _GROUPS, group_body, (s_acc, c_acc), unroll=4)

        s_acc, c_acc = lax.fori_loop(0, n_k, chunk_body, (zeros, zeros))
        s_tot = jnp.sum(s_acc)
        c_tot = jnp.sum(c_acc)
        res = jnp.where(lane == 0, s_tot, jnp.where(lane == 1, c_tot, 0.0))
        res_v[...] = res
        pltpu.sync_copy(res_v, out_hbm.at[wid])

    return body(tcat, rb)


def _finish(partials):
    def body(p_ref, o_ref):
        x = p_ref[...]
        s = jnp.sum(x[:, 0])
        c = jnp.sum(x[:, 1])
        o_ref[0] = s / jnp.maximum(c, 1.0)

    return pl.pallas_call(
        body,
        out_shape=jax.ShapeDtypeStruct((1,), jnp.float32),
        out_specs=pl.BlockSpec(memory_space=pltpu.SMEM),
    )(partials)


def kernel(sdf, all_edges):
    tcat = all_edges.reshape(-1, 128, 2).transpose(0, 2, 1).reshape(-1)
    tpk = lax.bitwise_or(lax.shift_right_logical(tcat, 1),
                         lax.shift_left(lax.bitwise_and(tcat, 1), 31))
    bits = lax.bitcast_convert_type(
        jnp.pad(sdf, (0, _NV_PAD - _N_VERTS)), jnp.uint32)
    rb = jnp.right_shift(
        bits + jnp.uint32(0x7FFF)
        + jnp.bitwise_and(jnp.right_shift(bits, jnp.uint32(16)),
                          jnp.uint32(1)),
        jnp.uint32(16)).astype(jnp.int32)
    partials = _sc_partials(tpk, rb)
    return _finish(partials)[0]
